# baseline (speedup 1.0000x reference)
import functools

import jax
import jax.numpy as jnp
from jax import lax
from jax.experimental import pallas as pl
from jax.experimental.pallas import tpu as pltpu
from jax.experimental.pallas import tpu_sc as plsc

NC = 2
NS = 16
NW = NC * NS

B = 100000
D = 128
CHUNK = 128
CHUNKS_PER_W = 25
B_PAD = NW * CHUNKS_PER_W * CHUNK


def _gather_body(idx_hbm, table_hbm, out_hbm, idx_v, rows_v, sem):
    wid = lax.axis_index("s") * NC + lax.axis_index("c")
    cbase = wid * CHUNKS_PER_W
    pltpu.sync_copy(idx_hbm.at[wid], idx_v)
    for j in range(CHUNKS_PER_W):
        pltpu.async_copy(table_hbm.at[idx_v.at[j]], rows_v, sem).wait()
        pltpu.sync_copy(rows_v, out_hbm.at[pl.ds((cbase + j) * CHUNK, CHUNK)])


@jax.jit
def _gather(idx2d, table):
    mesh = plsc.VectorSubcoreMesh(core_axis_name="c", subcore_axis_name="s")
    run = functools.partial(
        pl.kernel,
        mesh=mesh,
        out_type=jax.ShapeDtypeStruct((B_PAD, D), jnp.float32),
        scratch_types=[
            pltpu.VMEM((CHUNKS_PER_W, CHUNK), jnp.int32),
            pltpu.VMEM((CHUNK, D), jnp.float32),
            pltpu.SemaphoreType.DMA,
        ],
    )(_gather_body)
    return run(idx2d, table)


def kernel(f_nodes, f_edges, node2edge, edge2node, b2revb, edit_embedding):
    idx = jnp.pad(f_nodes.astype(jnp.int32), (0, B_PAD - B))
    idx2d = idx.reshape(NW, CHUNKS_PER_W, CHUNK)
    out = _gather(idx2d, edit_embedding)
    return out[:B]


# speedup vs baseline: 1.1538x; 1.1538x over previous
import functools

import jax
import jax.numpy as jnp
from jax import lax
from jax.experimental import pallas as pl
from jax.experimental.pallas import tpu as pltpu
from jax.experimental.pallas import tpu_sc as plsc

NC = 2
NS = 16
NW = NC * NS

B = 100000
D = 128
CHUNK = 128
CHUNKS_PER_W = 25
B_PAD = NW * CHUNKS_PER_W * CHUNK


RING = 5


def _gather_body(idx_hbm, table_hbm, out_hbm, idx_v, rows_v, gsem):
    wid = lax.axis_index("s") * NC + lax.axis_index("c")
    cbase = wid * CHUNKS_PER_W
    pltpu.sync_copy(idx_hbm.at[wid], idx_v)
    gathers = {}
    for j in range(RING):
        gathers[j] = pltpu.async_copy(
            table_hbm.at[idx_v.at[j]], rows_v.at[j], gsem.at[j])
    for j in range(CHUNKS_PER_W):
        b = j % RING
        gathers[j].wait()
        pltpu.sync_copy(rows_v.at[b], out_hbm.at[pl.ds((cbase + j) * CHUNK, CHUNK)])
        nj = j + RING
        if nj < CHUNKS_PER_W:
            gathers[nj] = pltpu.async_copy(
                table_hbm.at[idx_v.at[nj]], rows_v.at[b], gsem.at[b])


@jax.jit
def _gather(idx2d, table):
    mesh = plsc.VectorSubcoreMesh(core_axis_name="c", subcore_axis_name="s")
    run = functools.partial(
        pl.kernel,
        mesh=mesh,
        out_type=jax.ShapeDtypeStruct((B_PAD, D), jnp.float32),
        scratch_types=[
            pltpu.VMEM((CHUNKS_PER_W, CHUNK), jnp.int32),
            pltpu.VMEM((RING, CHUNK, D), jnp.float32),
            pltpu.SemaphoreType.DMA((RING,)),
        ],
    )(_gather_body)
    return run(idx2d, table)


def kernel(f_nodes, f_edges, node2edge, edge2node, b2revb, edit_embedding):
    idx = jnp.pad(f_nodes.astype(jnp.int32), (0, B_PAD - B))
    idx2d = idx.reshape(NW, CHUNKS_PER_W, CHUNK)
    out = _gather(idx2d, edit_embedding)
    return out[:B]


# speedup vs baseline: 3.7820x; 3.2777x over previous
import functools

import jax
import jax.numpy as jnp
from jax import lax
from jax.experimental import pallas as pl
from jax.experimental.pallas import tpu as pltpu
from jax.experimental.pallas import tpu_sc as plsc

NC = 2
NS = 16
NW = NC * NS

B = 100000
D = 128
PER_W = 3128
CHUNK = 128
FULL_CHUNKS = PER_W // CHUNK
TAIL = PER_W - FULL_CHUNKS * CHUNK
LAST_BASE = B - PER_W
RING = 6


def _gather_body(idx_hbm, table_hbm, out_hbm, idx_v, rows_v, tail_v, gsem, tsem):
    wid = lax.axis_index("s") * NC + lax.axis_index("c")
    base = pl.multiple_of(jnp.minimum(wid * PER_W, LAST_BASE), 8)
    pltpu.sync_copy(idx_hbm.at[pl.ds(base, PER_W)], idx_v)
    tail = pltpu.async_copy(
        table_hbm.at[idx_v.at[pl.ds(FULL_CHUNKS * CHUNK, TAIL)]], tail_v, tsem)
    gathers = {}
    for j in range(RING):
        gathers[j] = pltpu.async_copy(
            table_hbm.at[idx_v.at[pl.ds(j * CHUNK, CHUNK)]], rows_v.at[j],
            gsem.at[j])
    for j in range(FULL_CHUNKS):
        b = j % RING
        gathers[j].wait()
        pltpu.sync_copy(rows_v.at[b], out_hbm.at[pl.ds(base + j * CHUNK, CHUNK)])
        nj = j + RING
        if nj < FULL_CHUNKS:
            gathers[nj] = pltpu.async_copy(
                table_hbm.at[idx_v.at[pl.ds(nj * CHUNK, CHUNK)]], rows_v.at[b],
                gsem.at[b])
    tail.wait()
    pltpu.sync_copy(tail_v, out_hbm.at[pl.ds(base + FULL_CHUNKS * CHUNK, TAIL)])


@jax.jit
def _gather(idx, table):
    mesh = plsc.VectorSubcoreMesh(core_axis_name="c", subcore_axis_name="s")
    run = functools.partial(
        pl.kernel,
        mesh=mesh,
        out_type=jax.ShapeDtypeStruct((B, D), jnp.float32),
        scratch_types=[
            pltpu.VMEM((PER_W,), jnp.int32),
            pltpu.VMEM((RING, CHUNK, D), jnp.float32),
            pltpu.VMEM((TAIL, D), jnp.float32),
            pltpu.SemaphoreType.DMA((RING,)),
            pltpu.SemaphoreType.DMA,
        ],
    )(_gather_body)
    return run(idx, table)


def kernel(f_nodes, f_edges, node2edge, edge2node, b2revb, edit_embedding):
    return _gather(f_nodes.astype(jnp.int32), edit_embedding)
